# baseline (speedup 1.0000x reference)
import jax
import jax.numpy as jnp
from jax.experimental import pallas as pl

_HIDDEN = 2048
_HALF = _HIDDEN // 2
_NE = 8


def _router_body(x0_ref, x1_ref, x2_ref, x3_ref, w1_ref, b1_ref, w2_ref,
                 b2_ref, lt_ref, idx_ref):
    for b, x_ref in enumerate((x0_ref, x1_ref, x2_ref, x3_ref)):
        h = jnp.dot(x_ref[:], w1_ref[:], preferred_element_type=jnp.float32)
        h = jnp.maximum(h + b1_ref[:], 0.0)
        lt = jax.lax.dot_general(
            w2_ref[:], h, (((1,), (1,)), ((), ())),
            preferred_element_type=jnp.float32,
        ) + b2_ref[:]
        lt_ref[b] = lt
        m = jnp.max(lt, axis=0, keepdims=True)
        expert = jax.lax.broadcasted_iota(jnp.int32, lt.shape, 0)
        idx_ref[b] = jnp.min(jnp.where(lt == m, expert, _NE), axis=0)


def kernel(hidden_states, W1, b1, W2, b2):
    B, S, H = hidden_states.shape
    M = B * S
    x = hidden_states.reshape(M, H)
    m_blk = 256
    n_steps = S // m_blk
    grid = (n_steps,)

    def x_spec(b):
        return pl.BlockSpec((m_blk, H), lambda j, b=b: (b * n_steps + j, 0))

    lt, idx = pl.pallas_call(
        _router_body,
        grid=grid,
        in_specs=[
            x_spec(0),
            x_spec(1),
            x_spec(2),
            x_spec(3),
            pl.BlockSpec((H, _HALF), lambda j: (0, 0)),
            pl.BlockSpec((1, _HALF), lambda j: (0, 0)),
            pl.BlockSpec((_NE, _HALF), lambda j: (0, 0)),
            pl.BlockSpec((_NE, 1), lambda j: (0, 0)),
        ],
        out_specs=[
            pl.BlockSpec((B, _NE, m_blk), lambda j: (0, 0, j)),
            pl.BlockSpec((B, m_blk), lambda j: (0, j)),
        ],
        out_shape=[
            jax.ShapeDtypeStruct((B, _NE, S), jnp.float32),
            jax.ShapeDtypeStruct((B, S), jnp.int32),
        ],
    )(x, x, x, x, W1, b1.reshape(1, _HALF), W2.T, b2.reshape(_NE, 1))

    return idx, jnp.transpose(lt, (0, 2, 1))


# speedup vs baseline: 1.0212x; 1.0212x over previous
import jax
import jax.numpy as jnp
from jax.experimental import pallas as pl

_HIDDEN = 2048
_HALF = _HIDDEN // 2
_NE = 8


def _router_body(x0_ref, x1_ref, x2_ref, x3_ref, w1_ref, b1_ref, w2_ref,
                 b2_ref, lt_ref, idx_ref):
    for b, x_ref in enumerate((x0_ref, x1_ref, x2_ref, x3_ref)):
        h = jnp.dot(x_ref[:], w1_ref[:], preferred_element_type=jnp.float32)
        h = jnp.maximum(h + b1_ref[:], 0.0)
        lt = jax.lax.dot_general(
            w2_ref[:], h, (((1,), (1,)), ((), ())),
            preferred_element_type=jnp.float32,
        ) + b2_ref[:]
        lt_ref[b] = lt
        m = jnp.max(lt, axis=0, keepdims=True)
        expert = jax.lax.broadcasted_iota(jnp.int32, lt.shape, 0)
        idx_ref[b] = jnp.min(jnp.where(lt == m, expert, _NE), axis=0)


def kernel(hidden_states, W1, b1, W2, b2):
    B, S, H = hidden_states.shape
    M = B * S
    x = hidden_states.reshape(M, H)
    m_blk = 512
    n_steps = S // m_blk
    grid = (n_steps,)

    def x_spec(b):
        return pl.BlockSpec((m_blk, H), lambda j, b=b: (b * n_steps + j, 0))

    lt, idx = pl.pallas_call(
        _router_body,
        grid=grid,
        in_specs=[
            x_spec(0),
            x_spec(1),
            x_spec(2),
            x_spec(3),
            pl.BlockSpec((H, _HALF), lambda j: (0, 0)),
            pl.BlockSpec((1, _HALF), lambda j: (0, 0)),
            pl.BlockSpec((_NE, _HALF), lambda j: (0, 0)),
            pl.BlockSpec((_NE, 1), lambda j: (0, 0)),
        ],
        out_specs=[
            pl.BlockSpec((B, _NE, m_blk), lambda j: (0, 0, j)),
            pl.BlockSpec((B, m_blk), lambda j: (0, j)),
        ],
        out_shape=[
            jax.ShapeDtypeStruct((B, _NE, S), jnp.float32),
            jax.ShapeDtypeStruct((B, S), jnp.int32),
        ],
    )(x, x, x, x, W1, b1.reshape(1, _HALF), W2.T, b2.reshape(_NE, 1))

    return idx, jnp.transpose(lt, (0, 2, 1))


# speedup vs baseline: 1.0630x; 1.0410x over previous
import jax
import jax.numpy as jnp
from jax.experimental import pallas as pl

_HIDDEN = 2048
_HALF = _HIDDEN // 2
_NE = 8


def _router_body(x_ref, w1_ref, b1_ref, w2_ref, b2_ref, lt_ref, idx_ref):
    b = pl.program_id(1)
    h = jnp.dot(x_ref[:], w1_ref[:], preferred_element_type=jnp.float32)
    h = jnp.maximum(h + b1_ref[:], 0.0)
    lt = jax.lax.dot_general(
        w2_ref[:], h, (((1,), (1,)), ((), ())),
        preferred_element_type=jnp.float32,
    ) + b2_ref[:]
    lt_ref[b] = lt
    m = jnp.max(lt, axis=0, keepdims=True)
    expert = jax.lax.broadcasted_iota(jnp.int32, lt.shape, 0)
    idx = jnp.min(jnp.where(lt == m, expert, _NE), axis=0)
    row = jax.lax.broadcasted_iota(jnp.int32, idx_ref.shape, 0)
    idx_ref[:] = jnp.where(row == b, idx[None, :], idx_ref[:])


def kernel(hidden_states, W1, b1, W2, b2):
    B, S, H = hidden_states.shape
    M = B * S
    x = hidden_states.reshape(M, H)
    m_blk = 1024
    n_chunks = S // m_blk

    lt, idx = pl.pallas_call(
        _router_body,
        grid=(n_chunks, B),
        in_specs=[
            pl.BlockSpec((m_blk, H), lambda j, b: (b * n_chunks + j, 0)),
            pl.BlockSpec((H, _HALF), lambda j, b: (0, 0)),
            pl.BlockSpec((1, _HALF), lambda j, b: (0, 0)),
            pl.BlockSpec((_NE, _HALF), lambda j, b: (0, 0)),
            pl.BlockSpec((_NE, 1), lambda j, b: (0, 0)),
        ],
        out_specs=[
            pl.BlockSpec((B, _NE, m_blk), lambda j, b: (0, 0, j)),
            pl.BlockSpec((B, m_blk), lambda j, b: (0, j)),
        ],
        out_shape=[
            jax.ShapeDtypeStruct((B, _NE, S), jnp.float32),
            jax.ShapeDtypeStruct((B, S), jnp.int32),
        ],
    )(x, W1, b1.reshape(1, _HALF), W2.T, b2.reshape(_NE, 1))

    return idx, jnp.transpose(lt, (0, 2, 1))
